# baseline (speedup 1.0000x reference)
import jax
import jax.numpy as jnp
from jax.experimental import pallas as pl
from jax.experimental.pallas import tpu as pltpu
from jax.experimental.pallas import tpu_sc as plsc

_WINDOW = 256


def kernel(model_input, table):
    batch, seq = model_input.shape
    num_idx = batch * seq
    rows, dim = table.shape
    flat_idx = model_input.reshape(num_idx)
    pair_idx = (flat_idx >> 1).reshape(1, num_idx)
    pair_table = table.reshape(rows // 2, dim * 2)

    mesh = plsc.VectorSubcoreMesh(core_axis_name="core",
                                  subcore_axis_name="subcore")

    @pl.kernel(
        out_type=jax.ShapeDtypeStruct((num_idx, 2 * dim), table.dtype),
        mesh=mesh,
    )
    def gather(tab_hbm, idx_hbm, out_hbm):
        def body(idx_vmem, out_vmem):
            pltpu.sync_copy(tab_hbm.at[idx_vmem.at[0]], out_vmem)

        pltpu.emit_pipeline(
            body,
            grid=(num_idx // _WINDOW,),
            in_specs=[pl.BlockSpec((1, _WINDOW),
                                   index_map=lambda i: (0, i))],
            out_specs=[pl.BlockSpec((_WINDOW, 2 * dim),
                                    index_map=lambda i: (i, 0))],
            core_axis_name=("core", "subcore"),
            dimension_semantics=(pltpu.PARALLEL,),
        )(idx_hbm, out_hbm)

    pairs = gather(pair_table, pair_idx)
    odd = (flat_idx & 1).astype(bool)
    out = jnp.where(odd[:, None], pairs[:, dim:], pairs[:, :dim])
    return out.reshape(batch, seq, dim)


# speedup vs baseline: 1.1735x; 1.1735x over previous
import jax
import jax.numpy as jnp
from jax.experimental import pallas as pl
from jax.experimental.pallas import tpu as pltpu
from jax.experimental.pallas import tpu_sc as plsc

_WINDOW = 256


def kernel(model_input, table):
    batch, seq = model_input.shape
    num_idx = batch * seq
    rows, dim = table.shape
    indices = model_input.reshape(1, num_idx)

    padded = jnp.pad(table, ((0, 0), (0, 128 - dim)))

    mesh = plsc.VectorSubcoreMesh(core_axis_name="core",
                                  subcore_axis_name="subcore")

    @pl.kernel(
        out_type=jax.ShapeDtypeStruct((num_idx, dim), table.dtype),
        mesh=mesh,
        scratch_types=[pltpu.VMEM((_WINDOW, 128), jnp.float32)],
    )
    def gather(tab_hbm, idx_hbm, out_hbm, scratch):
        def body(idx_vmem, out_vmem):
            pltpu.sync_copy(tab_hbm.at[idx_vmem.at[0]], scratch)

            @pl.loop(0, _WINDOW)
            def _(k):
                src = scratch.at[k]
                dst = out_vmem.at[k]
                for c in range(0, dim, 16):
                    dst[pl.ds(c, 16)] = src[pl.ds(c, 16)]

        pltpu.emit_pipeline(
            body,
            grid=(num_idx // _WINDOW,),
            in_specs=[pl.BlockSpec((1, _WINDOW),
                                   index_map=lambda i: (0, i))],
            out_specs=[pl.BlockSpec((_WINDOW, dim),
                                    index_map=lambda i: (i, 0))],
            core_axis_name=("core", "subcore"),
            dimension_semantics=(pltpu.PARALLEL,),
        )(idx_hbm, out_hbm)

    out = gather(padded, indices)
    return out.reshape(batch, seq, dim)
